# Initial kernel scaffold; baseline (speedup 1.0000x reference)
import jax
import jax.numpy as jnp
from jax.experimental import pallas as pl


def kernel(value, W_k, b_k, w_q):
    raise NotImplementedError("write your pallas kernel here")


# speedup vs baseline: 3.7982x; 3.7982x over previous
import numpy as np
import jax
import jax.numpy as jnp
from jax.experimental import pallas as pl
from jax.experimental.pallas import tpu as pltpu

_TB = 256
_LN = 128


def _body(v_ref, wt_ref, wq_ref, lt_ref, bk_ref, o_ref, m_ref, den_ref, num_ref):
    i = pl.program_id(1)
    tb = v_ref.shape[1]
    d = v_ref.shape[2]
    nchunk = d // _LN

    @pl.when(i == 0)
    def _():
        m_ref[...] = jnp.full(m_ref.shape, -1e30, jnp.float32)
        den_ref[...] = jnp.zeros(den_ref.shape, jnp.float32)
        num_ref[...] = jnp.zeros(num_ref.shape, jnp.float32)

    v = v_ref[0]
    x = jnp.dot(v.astype(jnp.bfloat16), wt_ref[...],
                preferred_element_type=jnp.float32)
    k_act = jnp.tanh(x + bk_ref[...])
    s_rep = jnp.dot(k_act.astype(jnp.bfloat16), wq_ref[...],
                    preferred_element_type=jnp.float32)

    m_old = m_ref[...]
    m_new = jnp.maximum(m_old, jnp.max(s_rep, axis=0, keepdims=True))
    alpha = jnp.exp(m_old - m_new)
    e_rep = jnp.exp(s_rep - m_new)

    dcum = jnp.dot(lt_ref[...], e_rep.astype(jnp.bfloat16),
                   preferred_element_type=jnp.float32)
    den_full = den_ref[...] * alpha + dcum
    recip = 1.0 / den_full

    ev = jnp.concatenate(
        [v[:, j * _LN:(j + 1) * _LN] * e_rep for j in range(nchunk)],
        axis=1).astype(jnp.bfloat16)
    cums = jnp.dot(lt_ref[...], ev, preferred_element_type=jnp.float32)
    num_sc = jnp.concatenate(
        [num_ref[:, j * _LN:(j + 1) * _LN] * alpha for j in range(nchunk)],
        axis=1)
    num_full = num_sc + cums
    for j in range(nchunk):
        sl = slice(j * _LN, (j + 1) * _LN)
        o_ref[0, :, sl] = num_full[:, sl] * recip

    m_ref[...] = m_new
    den_ref[...] = den_full[tb - 1:tb, :]
    num_ref[...] = num_full[tb - 1:tb, :]


def kernel(value, W_k, b_k, w_q):
    B, T, D = value.shape
    nt = T // _TB
    wt = W_k.T.astype(jnp.bfloat16)
    wq_rep = jnp.broadcast_to(w_q[:, None], (D, _LN)).astype(jnp.bfloat16)
    ltri = jnp.asarray(np.tril(np.ones((_TB, _TB), np.float32)),
                       dtype=jnp.bfloat16)
    bk2 = b_k[None, :]
    return pl.pallas_call(
        _body,
        grid=(B, nt),
        in_specs=[
            pl.BlockSpec((1, _TB, D), lambda b, i: (b, i, 0)),
            pl.BlockSpec((D, D), lambda b, i: (0, 0)),
            pl.BlockSpec((D, _LN), lambda b, i: (0, 0)),
            pl.BlockSpec((_TB, _TB), lambda b, i: (0, 0)),
            pl.BlockSpec((1, D), lambda b, i: (0, 0)),
        ],
        out_specs=pl.BlockSpec((1, _TB, D), lambda b, i: (b, i, 0)),
        out_shape=jax.ShapeDtypeStruct((B, T, D), jnp.float32),
        scratch_shapes=[
            pltpu.VMEM((1, _LN), jnp.float32),
            pltpu.VMEM((1, _LN), jnp.float32),
            pltpu.VMEM((1, D), jnp.float32),
        ],
        compiler_params=pltpu.CompilerParams(
            dimension_semantics=("parallel", "arbitrary"),
        ),
        name="step_attention_fused",
    )(value, wt, wq_rep, ltri, bk2)


# speedup vs baseline: 4.2578x; 1.1210x over previous
import numpy as np
import jax
import jax.numpy as jnp
from jax.experimental import pallas as pl
from jax.experimental.pallas import tpu as pltpu

_TB = 256
_G = 64
_LN = 128


def _body(v_ref, wt_ref, lt_ref, bk_ref, wq_ref, ones_ref, o_ref,
          m_ref, den_ref, num_ref, s_scr, v_scr):
    i = pl.program_id(1)
    tb = v_ref.shape[1]
    d = v_ref.shape[2]
    nchunk = d // _LN
    ng = tb // _G

    s_rep = s_scr[...]
    vp = v_scr[...]

    @pl.when(i == 1)
    def _():
        m_ref[...] = jnp.full(m_ref.shape, -1e30, jnp.float32)
        den_ref[...] = jnp.zeros(den_ref.shape, jnp.float32)
        num_ref[...] = jnp.zeros(num_ref.shape, jnp.float32)

    m_old = m_ref[...]
    m_new = jnp.maximum(m_old, jnp.max(s_rep, axis=0, keepdims=True))
    alpha = jnp.exp(m_old - m_new)
    e_rep = jnp.exp(s_rep - m_new)
    e_bf = e_rep.astype(jnp.bfloat16)

    ev = jnp.concatenate(
        [vp[:, j * _LN:(j + 1) * _LN] * e_rep for j in range(nchunk)],
        axis=1).astype(jnp.bfloat16)
    nparts = []
    dparts = []
    for g in range(ng):
        rs = slice(g * _G, (g + 1) * _G)
        nparts.append(jnp.dot(lt_ref[...], ev[rs, :],
                              preferred_element_type=jnp.float32))
        dparts.append(jnp.dot(lt_ref[...], e_bf[rs, :],
                              preferred_element_type=jnp.float32))
    for g in range(1, ng):
        nparts[g] = nparts[g] + nparts[g - 1][_G - 1:_G, :]
        dparts[g] = dparts[g] + dparts[g - 1][_G - 1:_G, :]
    cums = jnp.concatenate(nparts, axis=0)
    dcum = jnp.concatenate(dparts, axis=0)

    den_full = den_ref[...] * alpha + dcum
    recip = 1.0 / den_full
    num_sc = jnp.concatenate(
        [num_ref[:, j * _LN:(j + 1) * _LN] * alpha for j in range(nchunk)],
        axis=1)
    num_full = num_sc + cums
    for j in range(nchunk):
        sl = slice(j * _LN, (j + 1) * _LN)
        o_ref[0, :, sl] = num_full[:, sl] * recip

    m_ref[...] = m_new
    den_ref[...] = den_full[tb - 1:tb, :]
    num_ref[...] = num_full[tb - 1:tb, :]

    v = v_ref[0]
    x = jnp.dot(v.astype(jnp.bfloat16), wt_ref[...],
                preferred_element_type=jnp.float32)
    k_act = jnp.tanh(x + bk_ref[...])
    s128 = k_act[:, 0:_LN] * wq_ref[0:1, :]
    for j in range(1, nchunk):
        s128 = s128 + k_act[:, j * _LN:(j + 1) * _LN] * wq_ref[j:j + 1, :]
    s_scr[...] = jnp.dot(s128.astype(jnp.bfloat16), ones_ref[...],
                         preferred_element_type=jnp.float32)
    v_scr[...] = v


def kernel(value, W_k, b_k, w_q):
    B, T, D = value.shape
    nt = T // _TB
    wt = W_k.T.astype(jnp.bfloat16)
    wq2 = w_q.reshape(D // _LN, _LN)
    ltri = jnp.asarray(np.tril(np.ones((_G, _G), np.float32)),
                       dtype=jnp.bfloat16)
    ones128 = jnp.ones((_LN, _LN), dtype=jnp.bfloat16)
    bk2 = b_k[None, :]
    return pl.pallas_call(
        _body,
        grid=(B, nt + 1),
        in_specs=[
            pl.BlockSpec((1, _TB, D),
                         lambda b, i: (b, jnp.minimum(i, nt - 1), 0)),
            pl.BlockSpec((D, D), lambda b, i: (0, 0)),
            pl.BlockSpec((_G, _G), lambda b, i: (0, 0)),
            pl.BlockSpec((1, D), lambda b, i: (0, 0)),
            pl.BlockSpec((D // _LN, _LN), lambda b, i: (0, 0)),
            pl.BlockSpec((_LN, _LN), lambda b, i: (0, 0)),
        ],
        out_specs=pl.BlockSpec((1, _TB, D),
                               lambda b, i: (b, jnp.maximum(i - 1, 0), 0)),
        out_shape=jax.ShapeDtypeStruct((B, T, D), jnp.float32),
        scratch_shapes=[
            pltpu.VMEM((1, _LN), jnp.float32),
            pltpu.VMEM((1, _LN), jnp.float32),
            pltpu.VMEM((1, D), jnp.float32),
            pltpu.VMEM((_TB, _LN), jnp.float32),
            pltpu.VMEM((_TB, D), jnp.float32),
        ],
        compiler_params=pltpu.CompilerParams(
            dimension_semantics=("parallel", "arbitrary"),
        ),
        name="step_attention_fused",
    )(value, wt, ltri, bk2, wq2, ones128)
